# baseline (speedup 1.0000x reference)
import jax
import jax.numpy as jnp
from jax import lax
from jax.experimental import pallas as pl
from jax.experimental.pallas import tpu as pltpu
from jax.experimental.pallas import tpu_sc as plsc

DICT_LEN = 100000
D = 64
NG = 5
NC, NS, L = 2, 16, 16
NW = NC * NS
STRIDE = 3120
ROWS_W = DICT_LEN - (NW - 1) * STRIDE
CHUNK = 80
NCH = ROWS_W // CHUNK
G = 5

_DN = (((1,), (1,)), ((), ()))


def _sc_retrieve(keys_mem, cue):
    mesh = plsc.VectorSubcoreMesh(core_axis_name="c", subcore_axis_name="s")

    def body(keys_hbm, cue_hbm, out_d, out_i, cue_v, keys_v,
             resd_v, resi_v, sem0, sem1):
        c = lax.axis_index("c")
        s = lax.axis_index("s")
        wid = s * NC + c
        base = wid * STRIDE

        pltpu.sync_copy(cue_hbm, cue_v)
        cues = [cue_v[pl.ds(q * L, L)] for q in range(D // L)]
        sems = (sem0, sem1)

        def start(ch):
            return pltpu.async_copy(
                keys_hbm.at[pl.ds(base + ch * CHUNK, CHUNK)],
                keys_v.at[ch % 2], sems[ch % 2])

        handles = {0: start(0)}
        bd = jnp.float32(jnp.inf)
        bi = jnp.int32(0)
        for ch in range(NCH):
            if ch + 1 < NCH:
                handles[ch + 1] = start(ch + 1)
            handles[ch].wait()
            cb = base + ch * CHUNK
            buf = ch % 2

            def group(g, carry, buf=buf, cb=cb):
                gd, gi = carry
                rb = g * G
                for j in range(G):
                    r = rb + j
                    acc = None
                    for q in range(D // L):
                        dq = keys_v[buf, r, pl.ds(q * L, L)] - cues[q]
                        sq = dq * dq
                        acc = sq if acc is None else acc + sq
                    tot = jnp.sum(acc)
                    take = tot < gd
                    gd = jnp.where(take, tot, gd)
                    gi = jnp.where(take, cb + r, gi)
                return gd, gi

            bd, bi = lax.fori_loop(0, CHUNK // G, group, (bd, bi))

        resd_v[0] = jnp.full((L,), bd, jnp.float32)
        resi_v[0] = jnp.full((L,), bi, jnp.int32)
        pltpu.sync_copy(resd_v, out_d.at[wid])
        pltpu.sync_copy(resi_v, out_i.at[wid])

    f = pl.kernel(
        body,
        compiler_params=pltpu.CompilerParams(needs_layout_passes=False),
        out_type=(
            jax.ShapeDtypeStruct((NW, 1, L), jnp.float32),
            jax.ShapeDtypeStruct((NW, 1, L), jnp.int32),
        ),
        mesh=mesh,
        scratch_types=[
            pltpu.VMEM((D,), jnp.float32),
            pltpu.VMEM((2, CHUNK, D), jnp.float32),
            pltpu.VMEM((1, L), jnp.float32),
            pltpu.VMEM((1, L), jnp.int32),
            pltpu.SemaphoreType.DMA,
            pltpu.SemaphoreType.DMA,
        ],
    )
    return f(keys_mem, cue)


def _merge_body(d2_ref, idx_ref, bi_ref):
    d2 = d2_ref[...]
    idx = idx_ref[...]
    mn = jnp.min(d2)
    big = jnp.int32(jnp.iinfo(jnp.int32).max)
    bi = jnp.min(jnp.where(d2 == mn, idx, big))
    bi_ref[...] = jnp.full((1, 1), bi, jnp.int32)


def _tc_body(bi_sref, vblk_ref, x_ref, h_ref, c_ref, wih_ref,
             whh_ref, bih_ref, bhh_ref, wa_ref, ba_ref, wc_ref, bc_ref,
             act_ref, val_ref, h_out, c_out):
    off = bi_sref[0] % 8
    rsel = (lax.broadcasted_iota(jnp.int32, (8, 1), 0) == off)
    m_t = jnp.sum(vblk_ref[...] * rsel.astype(jnp.float32), axis=0,
                  keepdims=True)

    x = x_ref[...]
    h = h_ref[...]
    pre = (lax.dot_general(x, wih_ref[...], _DN,
                           precision=lax.Precision.HIGHEST,
                           preferred_element_type=jnp.float32) +
           lax.dot_general(h, whh_ref[...], _DN,
                           precision=lax.Precision.HIGHEST,
                           preferred_element_type=jnp.float32) +
           bih_ref[...] + bhh_ref[...])
    i_t = jax.nn.sigmoid(pre[:, 0 * D:1 * D])
    f_t = jax.nn.sigmoid(pre[:, 1 * D:2 * D])
    g_t = jnp.tanh(pre[:, 2 * D:3 * D])
    o_t = jax.nn.sigmoid(pre[:, 3 * D:4 * D])
    r_t = jax.nn.sigmoid(pre[:, 4 * D:5 * D])
    c_t = f_t * c_ref[...] + i_t * g_t + r_t * m_t
    h_t = o_t * jnp.tanh(c_t)

    logits = lax.dot_general(h_t, wa_ref[...], _DN,
                             precision=lax.Precision.HIGHEST,
                             preferred_element_type=jnp.float32)
    logits = logits + ba_ref[...]
    act_ref[...] = jax.nn.softmax(logits, axis=-1)
    val_ref[...] = lax.dot_general(h_t, wc_ref[...], _DN,
                                   precision=lax.Precision.HIGHEST,
                                   preferred_element_type=jnp.float32)
    val_ref[...] += bc_ref[...]
    h_out[...] = h_t
    c_out[...] = c_t


def kernel(state, p_action, p_reward, timestep, cue, h_prev, c_prev, keys_mem,
           vals_mem, W_ih, W_hh, b_ih, b_hh, W_actor, b_actor, W_critic,
           b_critic):
    d2c, idxc = _sc_retrieve(keys_mem, cue)

    x_row = jnp.concatenate([state, p_action, p_reward, timestep],
                            axis=-1).reshape(1, D)

    bi_arr = pl.pallas_call(
        _merge_body,
        out_shape=jax.ShapeDtypeStruct((1, 1), jnp.int32),
    )(d2c.reshape(NW, L), idxc.reshape(NW, L))

    full = lambda shp: pl.BlockSpec(shp, lambda i, bi_ref: tuple(0 for _ in shp))
    act, val, h_t, c_t = pl.pallas_call(
        _tc_body,
        grid_spec=pltpu.PrefetchScalarGridSpec(
            num_scalar_prefetch=1,
            grid=(1,),
            in_specs=[
                pl.BlockSpec((8, D), lambda i, bi_ref: (bi_ref[0] // 8, 0)),
                full((1, D)), full((1, D)), full((1, D)),
                full((NG * D, D)), full((NG * D, D)),
                full((1, NG * D)), full((1, NG * D)),
                full((16, D)), full((1, 16)), full((1, D)), full((1, 1)),
            ],
            out_specs=[full((1, 16)), full((1, 1)), full((1, D)),
                       full((1, D))],
        ),
        out_shape=(
            jax.ShapeDtypeStruct((1, 16), jnp.float32),
            jax.ShapeDtypeStruct((1, 1), jnp.float32),
            jax.ShapeDtypeStruct((1, D), jnp.float32),
            jax.ShapeDtypeStruct((1, D), jnp.float32),
        ),
    )(bi_arr.reshape(1), vals_mem, x_row,
      h_prev.reshape(1, D), c_prev.reshape(1, D), W_ih, W_hh,
      b_ih.reshape(1, NG * D), b_hh.reshape(1, NG * D), W_actor,
      b_actor.reshape(1, 16), W_critic, b_critic.reshape(1, 1))

    return (act.reshape(16), val.reshape(1), h_t.reshape(D), c_t.reshape(D))


# speedup vs baseline: 2.2079x; 2.2079x over previous
import jax
import jax.numpy as jnp
from jax import lax
from jax.experimental import pallas as pl
from jax.experimental.pallas import tpu as pltpu
from jax.experimental.pallas import tpu_sc as plsc

DICT_LEN = 100000
D = 64
NG = 5
NC, NS, L = 2, 16, 16
NW = NC * NS
BLK = 128
NBLK = DICT_LEN // BLK
TAIL = DICT_LEN - NBLK * BLK
BPW = (NBLK + NW - 1) // NW
NGRP = BLK // L


def _sc_retrieve(keys_t, cue):
    mesh = plsc.VectorSubcoreMesh(core_axis_name="c", subcore_axis_name="s")

    def body(keys_hbm, cue_hbm, out_d, out_i, cue_v, keys_v, resd_v, resi_v,
             sem0, sem1):
        c = lax.axis_index("c")
        s = lax.axis_index("s")
        wid = s * NC + c

        pltpu.sync_copy(cue_hbm, cue_v)
        sems = (sem0, sem1)
        lane = lax.iota(jnp.int32, L)

        def blk_of(k):
            b = wid + NW * k
            b = jnp.where(b >= NBLK, b - NBLK, b)
            return b

        def start(k):
            cb = pl.multiple_of(blk_of(k) * BLK, BLK)
            return pltpu.async_copy(keys_hbm.at[:, pl.ds(cb, BLK)],
                                    keys_v.at[k % 2], sems[k % 2])

        handles = {0: start(0)}
        inf = jnp.float32(jnp.inf)
        rmin = [jnp.full((L,), inf) for _ in range(NGRP)]
        ridx = [jnp.zeros((L,), jnp.int32) for _ in range(NGRP)]
        for k in range(BPW):
            if k + 1 < BPW:
                handles[k + 1] = start(k + 1)
            handles[k].wait()
            buf = k % 2

            def dim_step(d, accs, buf=buf):
                cs = plsc.load_gather(cue_v, [jnp.full((L,), d, jnp.int32)])
                out = []
                for g in range(NGRP):
                    dq = keys_v[buf, d, pl.ds(g * L, L)] - cs
                    out.append(accs[g] + dq * dq)
                return tuple(out)

            accs = lax.fori_loop(0, D, dim_step,
                                 tuple(jnp.zeros((L,)) for _ in range(NGRP)))
            cb = blk_of(k) * BLK
            for g in range(NGRP):
                col = lane + (cb + g * L)
                take = accs[g] < rmin[g]
                rmin[g] = jnp.where(take, accs[g], rmin[g])
                ridx[g] = jnp.where(take, col, ridx[g])

        fd, fi = rmin[0], ridx[0]
        for g in range(1, NGRP):
            take = rmin[g] < fd
            tie = (rmin[g] == fd) & (ridx[g] < fi)
            upd = take | tie
            fd = jnp.where(upd, rmin[g], fd)
            fi = jnp.where(upd, ridx[g], fi)

        resd_v[0] = fd
        resi_v[0] = fi
        pltpu.sync_copy(resd_v, out_d.at[wid])
        pltpu.sync_copy(resi_v, out_i.at[wid])

    f = pl.kernel(
        body,
        compiler_params=pltpu.CompilerParams(needs_layout_passes=False),
        out_type=(
            jax.ShapeDtypeStruct((NW, 1, L), jnp.float32),
            jax.ShapeDtypeStruct((NW, 1, L), jnp.int32),
        ),
        mesh=mesh,
        scratch_types=[
            pltpu.VMEM((D,), jnp.float32),
            pltpu.VMEM((2, D, BLK), jnp.float32),
            pltpu.VMEM((1, L), jnp.float32),
            pltpu.VMEM((1, L), jnp.int32),
            pltpu.SemaphoreType.DMA,
            pltpu.SemaphoreType.DMA,
        ],
    )
    return f(keys_t, cue)


def _merge_body(d2_ref, idx_ref, tail_ref, cue_ref, bi_ref):
    d2 = d2_ref[...]
    idx = idx_ref[...]
    dt = tail_ref[...] - cue_ref[...]
    d2t = jnp.sum(dt * dt, axis=0, keepdims=True)
    idxt = lax.broadcasted_iota(jnp.int32, (1, TAIL), 1) + (NBLK * BLK)
    mn = jnp.minimum(jnp.min(d2), jnp.min(d2t))
    big = jnp.int32(jnp.iinfo(jnp.int32).max)
    bi = jnp.minimum(jnp.min(jnp.where(d2 == mn, idx, big)),
                     jnp.min(jnp.where(d2t == mn, idxt, big)))
    bi_ref[...] = jnp.full((1, 1), bi, jnp.int32)


def _tc_body(bi_sref, vblk_ref, x_ref, h_ref, c_ref, wih_ref,
             whh_ref, bih_ref, bhh_ref, wa_ref, ba_ref, wc_ref, bc_ref,
             act_ref, val_ref, h_out, c_out):
    off = bi_sref[0] % BLK
    csel = (lax.broadcasted_iota(jnp.int32, (1, BLK), 1) == off)
    m_col = jnp.sum(vblk_ref[...] * csel.astype(jnp.float32), axis=1,
                    keepdims=True)
    eye = (lax.broadcasted_iota(jnp.int32, (D, D), 0) ==
           lax.broadcasted_iota(jnp.int32, (D, D), 1)).astype(jnp.float32)
    m_t = lax.dot_general(m_col, eye, (((0,), (0,)), ((), ())),
                          precision=lax.Precision.HIGHEST,
                          preferred_element_type=jnp.float32)

    x = x_ref[...]
    h = h_ref[...]
    dn = (((1,), (0,)), ((), ()))
    pre = (lax.dot_general(x, wih_ref[...], dn,
                           precision=lax.Precision.HIGHEST,
                           preferred_element_type=jnp.float32) +
           lax.dot_general(h, whh_ref[...], dn,
                           precision=lax.Precision.HIGHEST,
                           preferred_element_type=jnp.float32) +
           bih_ref[...] + bhh_ref[...])
    i_t = jax.nn.sigmoid(pre[:, 0 * D:1 * D])
    f_t = jax.nn.sigmoid(pre[:, 1 * D:2 * D])
    g_t = jnp.tanh(pre[:, 2 * D:3 * D])
    o_t = jax.nn.sigmoid(pre[:, 3 * D:4 * D])
    r_t = jax.nn.sigmoid(pre[:, 4 * D:5 * D])
    c_t = f_t * c_ref[...] + i_t * g_t + r_t * m_t
    h_t = o_t * jnp.tanh(c_t)

    logits = lax.dot_general(h_t, wa_ref[...], dn,
                             precision=lax.Precision.HIGHEST,
                             preferred_element_type=jnp.float32)
    logits = logits + ba_ref[...]
    act_ref[...] = jax.nn.softmax(logits, axis=-1)
    val_ref[...] = lax.dot_general(h_t, wc_ref[...], dn,
                                   precision=lax.Precision.HIGHEST,
                                   preferred_element_type=jnp.float32)
    val_ref[...] += bc_ref[...]
    h_out[...] = h_t
    c_out[...] = c_t


def kernel(state, p_action, p_reward, timestep, cue, h_prev, c_prev, keys_mem,
           vals_mem, W_ih, W_hh, b_ih, b_hh, W_actor, b_actor, W_critic,
           b_critic):
    keys_t = keys_mem.T
    vals_t = vals_mem.T
    d2c, idxc = _sc_retrieve(keys_t, cue)

    x_row = jnp.concatenate([state, p_action, p_reward, timestep],
                            axis=-1).reshape(1, D)

    bi_arr = pl.pallas_call(
        _merge_body,
        out_shape=jax.ShapeDtypeStruct((1, 1), jnp.int32),
    )(d2c.reshape(NW, L), idxc.reshape(NW, L),
      lax.slice(keys_t, (0, NBLK * BLK), (D, DICT_LEN)), cue.reshape(D, 1))

    full = lambda shp: pl.BlockSpec(shp, lambda i, bi_ref: tuple(
        0 for _ in shp))
    act, val, h_t, c_t = pl.pallas_call(
        _tc_body,
        grid_spec=pltpu.PrefetchScalarGridSpec(
            num_scalar_prefetch=1,
            grid=(1,),
            in_specs=[
                pl.BlockSpec((D, BLK), lambda i, bi_ref: (0, bi_ref[0] // BLK)),
                full((1, D)), full((1, D)), full((1, D)),
                full((D, NG * D)), full((D, NG * D)),
                full((1, NG * D)), full((1, NG * D)),
                full((D, 16)), full((1, 16)), full((D, 1)), full((1, 1)),
            ],
            out_specs=[full((1, 16)), full((1, 1)), full((1, D)),
                       full((1, D))],
        ),
        out_shape=(
            jax.ShapeDtypeStruct((1, 16), jnp.float32),
            jax.ShapeDtypeStruct((1, 1), jnp.float32),
            jax.ShapeDtypeStruct((1, D), jnp.float32),
            jax.ShapeDtypeStruct((1, D), jnp.float32),
        ),
    )(bi_arr.reshape(1), vals_t, x_row,
      h_prev.reshape(1, D), c_prev.reshape(1, D), W_ih.T, W_hh.T,
      b_ih.reshape(1, NG * D), b_hh.reshape(1, NG * D), W_actor.T,
      b_actor.reshape(1, 16), W_critic.T, b_critic.reshape(1, 1))

    return (act.reshape(16), val.reshape(1), h_t.reshape(D), c_t.reshape(D))


# speedup vs baseline: 2.6309x; 1.1916x over previous
import jax
import jax.numpy as jnp
from jax import lax
from jax.experimental import pallas as pl
from jax.experimental.pallas import tpu as pltpu
from jax.experimental.pallas import tpu_sc as plsc

DICT_LEN = 100000
D = 64
NG = 5
NC, NS, L = 2, 16, 16
NW = NC * NS
BLK = 128
BPW = 12
NBLK = NW * BPW
C0 = NBLK * BLK
NGRP = BLK // L
TCB = 4096
TCG = (DICT_LEN - C0 + TCB - 1) // TCB


def _sc_retrieve(keys_t, cue):
    mesh = plsc.VectorSubcoreMesh(core_axis_name="c", subcore_axis_name="s")

    def body(keys_hbm, cue_hbm, out_d, out_i, cue_v, keys_v, resd_v, resi_v,
             sem0, sem1):
        c = lax.axis_index("c")
        s = lax.axis_index("s")
        wid = s * NC + c

        pltpu.sync_copy(cue_hbm, cue_v)
        sems = (sem0, sem1)
        lane = lax.iota(jnp.int32, L)

        def blk_of(k):
            return wid + NW * k

        def start(k):
            cb = pl.multiple_of(blk_of(k) * BLK, BLK)
            return pltpu.async_copy(keys_hbm.at[:, pl.ds(cb, BLK)],
                                    keys_v.at[k % 2], sems[k % 2])

        handles = {0: start(0)}
        inf = jnp.float32(jnp.inf)
        rmin = [jnp.full((L,), inf) for _ in range(NGRP)]
        ridx = [jnp.zeros((L,), jnp.int32) for _ in range(NGRP)]
        for k in range(BPW):
            if k + 1 < BPW:
                handles[k + 1] = start(k + 1)
            handles[k].wait()
            buf = k % 2

            def dim_step(d, accs, buf=buf):
                cs = plsc.load_gather(cue_v, [jnp.full((L,), d, jnp.int32)])
                out = []
                for g in range(NGRP):
                    dq = keys_v[buf, d, pl.ds(g * L, L)] - cs
                    out.append(accs[g] + dq * dq)
                return tuple(out)

            accs = lax.fori_loop(0, D, dim_step,
                                 tuple(jnp.zeros((L,)) for _ in range(NGRP)))
            cb = blk_of(k) * BLK
            for g in range(NGRP):
                col = lane + (cb + g * L)
                take = accs[g] < rmin[g]
                rmin[g] = jnp.where(take, accs[g], rmin[g])
                ridx[g] = jnp.where(take, col, ridx[g])

        fd, fi = rmin[0], ridx[0]
        for g in range(1, NGRP):
            take = rmin[g] < fd
            tie = (rmin[g] == fd) & (ridx[g] < fi)
            upd = take | tie
            fd = jnp.where(upd, rmin[g], fd)
            fi = jnp.where(upd, ridx[g], fi)

        resd_v[0] = fd
        resi_v[0] = fi
        pltpu.sync_copy(resd_v, out_d.at[wid])
        pltpu.sync_copy(resi_v, out_i.at[wid])

    f = pl.kernel(
        body,
        compiler_params=pltpu.CompilerParams(needs_layout_passes=False),
        out_type=(
            jax.ShapeDtypeStruct((NW, 1, L), jnp.float32),
            jax.ShapeDtypeStruct((NW, 1, L), jnp.int32),
        ),
        mesh=mesh,
        scratch_types=[
            pltpu.VMEM((D,), jnp.float32),
            pltpu.VMEM((2, D, BLK), jnp.float32),
            pltpu.VMEM((1, L), jnp.float32),
            pltpu.VMEM((1, L), jnp.int32),
            pltpu.SemaphoreType.DMA,
            pltpu.SemaphoreType.DMA,
        ],
    )
    return f(keys_t, cue)


def _tcd2_body(keys_ref, cue_ref, vmin_ref, vidx_ref):
    i = pl.program_id(0)
    blk = keys_ref[...] - cue_ref[...]
    d2 = jnp.sum(blk * blk, axis=0, keepdims=True)
    cols = lax.broadcasted_iota(jnp.int32, (1, TCB), 1) + (C0 + i * TCB)
    d2 = jnp.where(cols < DICT_LEN, d2, jnp.float32(jnp.inf))

    @pl.when(i == 0)
    def _():
        vmin_ref[...] = d2
        vidx_ref[...] = cols

    @pl.when(i > 0)
    def _():
        take = d2 < vmin_ref[...]
        vmin_ref[...] = jnp.where(take, d2, vmin_ref[...])
        vidx_ref[...] = jnp.where(take, cols, vidx_ref[...])


def _merge_body(d2_ref, idx_ref, tmin_ref, tidx_ref, bi_ref):
    d2 = d2_ref[...]
    idx = idx_ref[...]
    tm = tmin_ref[...]
    ti = tidx_ref[...]
    mn = jnp.minimum(jnp.min(d2), jnp.min(tm))
    big = jnp.int32(jnp.iinfo(jnp.int32).max)
    bi = jnp.minimum(jnp.min(jnp.where(d2 == mn, idx, big)),
                     jnp.min(jnp.where(tm == mn, ti, big)))
    bi_ref[...] = jnp.full((1, 1), bi, jnp.int32)


def _tc_body(bi_sref, vblk_ref, x_ref, h_ref, c_ref, wih_ref,
             whh_ref, bih_ref, bhh_ref, wa_ref, ba_ref, wc_ref, bc_ref,
             act_ref, val_ref, h_out, c_out):
    off = bi_sref[0] % BLK
    csel = (lax.broadcasted_iota(jnp.int32, (1, BLK), 1) == off)
    m_col = jnp.sum(vblk_ref[...] * csel.astype(jnp.float32), axis=1,
                    keepdims=True)
    eye = (lax.broadcasted_iota(jnp.int32, (D, D), 0) ==
           lax.broadcasted_iota(jnp.int32, (D, D), 1)).astype(jnp.float32)
    m_t = lax.dot_general(m_col, eye, (((0,), (0,)), ((), ())),
                          precision=lax.Precision.HIGHEST,
                          preferred_element_type=jnp.float32)

    x = x_ref[...]
    h = h_ref[...]
    dn = (((1,), (0,)), ((), ()))
    pre = (lax.dot_general(x, wih_ref[...], dn,
                           precision=lax.Precision.HIGHEST,
                           preferred_element_type=jnp.float32) +
           lax.dot_general(h, whh_ref[...], dn,
                           precision=lax.Precision.HIGHEST,
                           preferred_element_type=jnp.float32) +
           bih_ref[...] + bhh_ref[...])
    i_t = jax.nn.sigmoid(pre[:, 0 * D:1 * D])
    f_t = jax.nn.sigmoid(pre[:, 1 * D:2 * D])
    g_t = jnp.tanh(pre[:, 2 * D:3 * D])
    o_t = jax.nn.sigmoid(pre[:, 3 * D:4 * D])
    r_t = jax.nn.sigmoid(pre[:, 4 * D:5 * D])
    c_t = f_t * c_ref[...] + i_t * g_t + r_t * m_t
    h_t = o_t * jnp.tanh(c_t)

    logits = lax.dot_general(h_t, wa_ref[...], dn,
                             precision=lax.Precision.HIGHEST,
                             preferred_element_type=jnp.float32)
    logits = logits + ba_ref[...]
    act_ref[...] = jax.nn.softmax(logits, axis=-1)
    val_ref[...] = lax.dot_general(h_t, wc_ref[...], dn,
                                   precision=lax.Precision.HIGHEST,
                                   preferred_element_type=jnp.float32)
    val_ref[...] += bc_ref[...]
    h_out[...] = h_t
    c_out[...] = c_t


def kernel(state, p_action, p_reward, timestep, cue, h_prev, c_prev, keys_mem,
           vals_mem, W_ih, W_hh, b_ih, b_hh, W_actor, b_actor, W_critic,
           b_critic):
    keys_t = keys_mem.T
    vals_t = vals_mem.T
    d2c, idxc = _sc_retrieve(keys_t, cue)

    x_row = jnp.concatenate([state, p_action, p_reward, timestep],
                            axis=-1).reshape(1, D)

    tmin, tidx = pl.pallas_call(
        _tcd2_body,
        grid=(TCG,),
        in_specs=[
            pl.BlockSpec((D, TCB), lambda i: (0, C0 // TCB + i)),
            pl.BlockSpec((D, 1), lambda i: (0, 0)),
        ],
        out_specs=[pl.BlockSpec((1, TCB), lambda i: (0, 0)),
                   pl.BlockSpec((1, TCB), lambda i: (0, 0))],
        out_shape=(jax.ShapeDtypeStruct((1, TCB), jnp.float32),
                   jax.ShapeDtypeStruct((1, TCB), jnp.int32)),
    )(keys_t, cue.reshape(D, 1))

    bi_arr = pl.pallas_call(
        _merge_body,
        out_shape=jax.ShapeDtypeStruct((1, 1), jnp.int32),
    )(d2c.reshape(NW, L), idxc.reshape(NW, L), tmin, tidx)

    full = lambda shp: pl.BlockSpec(shp, lambda i, bi_ref: tuple(
        0 for _ in shp))
    act, val, h_t, c_t = pl.pallas_call(
        _tc_body,
        grid_spec=pltpu.PrefetchScalarGridSpec(
            num_scalar_prefetch=1,
            grid=(1,),
            in_specs=[
                pl.BlockSpec((D, BLK), lambda i, bi_ref: (0, bi_ref[0] // BLK)),
                full((1, D)), full((1, D)), full((1, D)),
                full((D, NG * D)), full((D, NG * D)),
                full((1, NG * D)), full((1, NG * D)),
                full((D, 16)), full((1, 16)), full((D, 1)), full((1, 1)),
            ],
            out_specs=[full((1, 16)), full((1, 1)), full((1, D)),
                       full((1, D))],
        ),
        out_shape=(
            jax.ShapeDtypeStruct((1, 16), jnp.float32),
            jax.ShapeDtypeStruct((1, 1), jnp.float32),
            jax.ShapeDtypeStruct((1, D), jnp.float32),
            jax.ShapeDtypeStruct((1, D), jnp.float32),
        ),
    )(bi_arr.reshape(1), vals_t, x_row,
      h_prev.reshape(1, D), c_prev.reshape(1, D), W_ih.T, W_hh.T,
      b_ih.reshape(1, NG * D), b_hh.reshape(1, NG * D), W_actor.T,
      b_actor.reshape(1, 16), W_critic.T, b_critic.reshape(1, 1))

    return (act.reshape(16), val.reshape(1), h_t.reshape(D), c_t.reshape(D))


# speedup vs baseline: 2.7344x; 1.0394x over previous
import jax
import jax.numpy as jnp
from jax import lax
from jax.experimental import pallas as pl
from jax.experimental.pallas import tpu as pltpu
from jax.experimental.pallas import tpu_sc as plsc

DICT_LEN = 100000
D = 64
NG = 5
NC, NS, L = 2, 16, 16
NW = NC * NS
BLK = 128
BPW = 12
NBLK = NW * BPW
C0 = NBLK * BLK
NGRP = BLK // L
TCB = 4096
TCG = (DICT_LEN - C0 + TCB - 1) // TCB


def _sc_retrieve(keys_t, cue):
    mesh = plsc.VectorSubcoreMesh(core_axis_name="c", subcore_axis_name="s")

    def body(keys_hbm, cue_hbm, out_d, out_i, cue_v, keys_v, resd_v, resi_v,
             sem0, sem1):
        c = lax.axis_index("c")
        s = lax.axis_index("s")
        wid = s * NC + c

        pltpu.sync_copy(cue_hbm, cue_v)
        sems = (sem0, sem1)
        lane = lax.iota(jnp.int32, L)

        def blk_of(k):
            return wid + NW * k

        def start(k):
            cb = pl.multiple_of(blk_of(k) * BLK, BLK)
            return pltpu.async_copy(keys_hbm.at[:, pl.ds(cb, BLK)],
                                    keys_v.at[k % 2], sems[k % 2])

        handles = {0: start(0)}
        inf = jnp.float32(jnp.inf)
        rmin = [jnp.full((L,), inf) for _ in range(NGRP)]
        ridx = [jnp.zeros((L,), jnp.int32) for _ in range(NGRP)]
        for k in range(BPW):
            if k + 1 < BPW:
                handles[k + 1] = start(k + 1)
            handles[k].wait()
            buf = k % 2

            def dim_step(d, accs, buf=buf):
                cs = plsc.load_gather(cue_v, [jnp.full((L,), d, jnp.int32)])
                out = []
                for g in range(NGRP):
                    dq = keys_v[buf, d, pl.ds(g * L, L)] - cs
                    out.append(accs[g] + dq * dq)
                return tuple(out)

            accs = lax.fori_loop(0, D, dim_step,
                                 tuple(jnp.zeros((L,)) for _ in range(NGRP)))
            cb = blk_of(k) * BLK
            for g in range(NGRP):
                col = lane + (cb + g * L)
                take = accs[g] < rmin[g]
                rmin[g] = jnp.where(take, accs[g], rmin[g])
                ridx[g] = jnp.where(take, col, ridx[g])

        fd, fi = rmin[0], ridx[0]
        for g in range(1, NGRP):
            take = rmin[g] < fd
            tie = (rmin[g] == fd) & (ridx[g] < fi)
            upd = take | tie
            fd = jnp.where(upd, rmin[g], fd)
            fi = jnp.where(upd, ridx[g], fi)

        resd_v[0] = fd
        resi_v[0] = fi
        pltpu.sync_copy(resd_v, out_d.at[wid])
        pltpu.sync_copy(resi_v, out_i.at[wid])

    f = pl.kernel(
        body,
        compiler_params=pltpu.CompilerParams(needs_layout_passes=False),
        out_type=(
            jax.ShapeDtypeStruct((NW, 1, L), jnp.float32),
            jax.ShapeDtypeStruct((NW, 1, L), jnp.int32),
        ),
        mesh=mesh,
        scratch_types=[
            pltpu.VMEM((D,), jnp.float32),
            pltpu.VMEM((2, D, BLK), jnp.float32),
            pltpu.VMEM((1, L), jnp.float32),
            pltpu.VMEM((1, L), jnp.int32),
            pltpu.SemaphoreType.DMA,
            pltpu.SemaphoreType.DMA,
        ],
    )
    return f(keys_t, cue)


def _tcd2_body(keys_ref, cue_ref, vmin_ref, vidx_ref):
    i = pl.program_id(0)
    blk = keys_ref[...] - cue_ref[...]
    d2 = jnp.sum(blk * blk, axis=0, keepdims=True)
    cols = lax.broadcasted_iota(jnp.int32, (1, TCB), 1) + (C0 + i * TCB)
    d2 = jnp.where(cols < DICT_LEN, d2, jnp.float32(jnp.inf))

    @pl.when(i == 0)
    def _():
        vmin_ref[...] = d2
        vidx_ref[...] = cols

    @pl.when(i > 0)
    def _():
        take = d2 < vmin_ref[...]
        vmin_ref[...] = jnp.where(take, d2, vmin_ref[...])
        vidx_ref[...] = jnp.where(take, cols, vidx_ref[...])


def _fin_body(d2_ref, idx_ref, tmin_ref, tidx_ref, vals_ref, st_ref,
              pa_ref, pr_ref, ts_ref, h_ref, c_ref, wih_ref, whh_ref,
              bih_ref, bhh_ref, wa_ref, ba_ref, wc_ref, bc_ref,
              act_ref, val_ref, h_out, c_out, vblk_ref, semg):
    d2 = d2_ref[:, 0, :]
    idx = idx_ref[:, 0, :]
    tm = tmin_ref[...]
    ti = tidx_ref[...]
    mn = jnp.minimum(jnp.min(d2), jnp.min(tm))
    big = jnp.int32(jnp.iinfo(jnp.int32).max)
    bi = jnp.minimum(jnp.min(jnp.where(d2 == mn, idx, big)),
                     jnp.min(jnp.where(tm == mn, ti, big)))

    cb = pl.multiple_of((bi // BLK) * BLK, BLK)
    cp = pltpu.make_async_copy(vals_ref.at[:, pl.ds(cb, BLK)], vblk_ref, semg)
    cp.start()
    cp.wait()
    csel = (lax.broadcasted_iota(jnp.int32, (1, BLK), 1) == (bi - cb))
    m_col = jnp.sum(vblk_ref[...] * csel.astype(jnp.float32), axis=1,
                    keepdims=True)
    eye = (lax.broadcasted_iota(jnp.int32, (D, D), 0) ==
           lax.broadcasted_iota(jnp.int32, (D, D), 1)).astype(jnp.float32)
    m_t = lax.dot_general(m_col, eye, (((0,), (0,)), ((), ())),
                          precision=lax.Precision.HIGHEST,
                          preferred_element_type=jnp.float32)

    x = jnp.concatenate([st_ref[...], pa_ref[...], pr_ref[...], ts_ref[...]],
                        axis=1)
    h = h_ref[...]
    dn = (((1,), (0,)), ((), ()))
    pre = (lax.dot_general(x, wih_ref[...], dn,
                           precision=lax.Precision.HIGHEST,
                           preferred_element_type=jnp.float32) +
           lax.dot_general(h, whh_ref[...], dn,
                           precision=lax.Precision.HIGHEST,
                           preferred_element_type=jnp.float32) +
           bih_ref[...] + bhh_ref[...])
    i_t = jax.nn.sigmoid(pre[:, 0 * D:1 * D])
    f_t = jax.nn.sigmoid(pre[:, 1 * D:2 * D])
    g_t = jnp.tanh(pre[:, 2 * D:3 * D])
    o_t = jax.nn.sigmoid(pre[:, 3 * D:4 * D])
    r_t = jax.nn.sigmoid(pre[:, 4 * D:5 * D])
    c_t = f_t * c_ref[...] + i_t * g_t + r_t * m_t
    h_t = o_t * jnp.tanh(c_t)

    logits = lax.dot_general(h_t, wa_ref[...], dn,
                             precision=lax.Precision.HIGHEST,
                             preferred_element_type=jnp.float32)
    logits = logits + ba_ref[...]
    act_ref[...] = jax.nn.softmax(logits, axis=-1)
    val_ref[...] = lax.dot_general(h_t, wc_ref[...], dn,
                                   precision=lax.Precision.HIGHEST,
                                   preferred_element_type=jnp.float32)
    val_ref[...] += bc_ref[...]
    h_out[...] = h_t
    c_out[...] = c_t


def kernel(state, p_action, p_reward, timestep, cue, h_prev, c_prev, keys_mem,
           vals_mem, W_ih, W_hh, b_ih, b_hh, W_actor, b_actor, W_critic,
           b_critic):
    keys_t = keys_mem.T
    vals_t = vals_mem.T
    d2c, idxc = _sc_retrieve(keys_t, cue)

    tmin, tidx = pl.pallas_call(
        _tcd2_body,
        grid=(TCG,),
        in_specs=[
            pl.BlockSpec((D, TCB), lambda i: (0, C0 // TCB + i)),
            pl.BlockSpec((D, 1), lambda i: (0, 0)),
        ],
        out_specs=[pl.BlockSpec((1, TCB), lambda i: (0, 0)),
                   pl.BlockSpec((1, TCB), lambda i: (0, 0))],
        out_shape=(jax.ShapeDtypeStruct((1, TCB), jnp.float32),
                   jax.ShapeDtypeStruct((1, TCB), jnp.int32)),
    )(keys_t, cue.reshape(D, 1))

    vm = pltpu.MemorySpace.VMEM
    specs = [pl.BlockSpec(memory_space=vm) for _ in range(19)]
    specs[4] = pl.BlockSpec(memory_space=pl.ANY)
    act, val, h_t, c_t = pl.pallas_call(
        _fin_body,
        in_specs=specs,
        scratch_shapes=[pltpu.VMEM((D, BLK), jnp.float32),
                        pltpu.SemaphoreType.DMA],
        out_shape=(
            jax.ShapeDtypeStruct((1, 16), jnp.float32),
            jax.ShapeDtypeStruct((1, 1), jnp.float32),
            jax.ShapeDtypeStruct((1, D), jnp.float32),
            jax.ShapeDtypeStruct((1, D), jnp.float32),
        ),
    )(d2c, idxc, tmin, tidx, vals_t, state.reshape(1, 61),
      p_action.reshape(1, 1), p_reward.reshape(1, 1), timestep.reshape(1, 1),
      h_prev.reshape(1, D), c_prev.reshape(1, D), W_ih.T, W_hh.T,
      b_ih.reshape(1, NG * D), b_hh.reshape(1, NG * D), W_actor.T,
      b_actor.reshape(1, 16), W_critic.T, b_critic.reshape(1, 1))

    return (act.reshape(16), val.reshape(1), h_t.reshape(D), c_t.reshape(D))
